# baseline (speedup 1.0000x reference)
import dataclasses

import jax
import jax.numpy as jnp
from jax import lax
from jax.experimental import pallas as pl
from jax.experimental.pallas import tpu as pltpu
from jax.experimental.pallas import tpu_sc as plsc

_BTILE = 128
_TILES = 32
_LANE = 128


def kernel(inputs, table):
    batch, hist = inputs.shape
    _, embed_dim = table.shape
    nchunk = embed_dim // 16
    pairs = hist // 2

    tab128 = jnp.pad(table, ((0, 0), (0, _LANE - embed_dim)))
    idx_t = inputs.T

    mesh = plsc.VectorSubcoreMesh(core_axis_name="c", subcore_axis_name="s")

    cp = pltpu.CompilerParams()
    if "needs_layout_passes" in pltpu.CompilerParams.__dataclass_fields__:
        cp = dataclasses.replace(cp, needs_layout_passes=False)

    @pl.kernel(
        out_type=jax.ShapeDtypeStruct((hist, embed_dim, batch), table.dtype),
        mesh=mesh,
        compiler_params=cp,
        scratch_types=[
            pltpu.VMEM((_BTILE,), jnp.int32),
            pltpu.VMEM((_BTILE,), jnp.int32),
            pltpu.VMEM((_BTILE, _LANE), jnp.float32),
            pltpu.VMEM((_BTILE, _LANE), jnp.float32),
            pltpu.VMEM((64, _BTILE), jnp.float32),
            pltpu.VMEM((64, _BTILE), jnp.float32),
            pltpu.SemaphoreType.DMA,
            pltpu.SemaphoreType.DMA,
            pltpu.SemaphoreType.DMA,
            pltpu.SemaphoreType.DMA,
            pltpu.SemaphoreType.DMA,
            pltpu.SemaphoreType.DMA,
        ],
    )
    def gather_kernel(table_hbm, idx_hbm, out_hbm,
                      idx_a, idx_b, gath_a, gath_b, outb_a, outb_b,
                      si_a, si_b, sg_a, sg_b, so_a, so_b):
        wid = lax.axis_index("c") * 16 + lax.axis_index("s")
        b0 = wid * _BTILE

        rows = [lax.iota(jnp.int32, 16) + 16 * c for c in range(nchunk)]

        def transpose_into(gath, outb):
            @pl.loop(0, _BTILE)
            def _(b):
                lanes = jnp.full((16,), b, jnp.int32)
                for c in range(nchunk):
                    x = gath.at[b, pl.ds(16 * c, 16)][...]
                    plsc.store_scatter(outb, [rows[c], lanes], x)

        pltpu.async_copy(idx_hbm.at[0, pl.ds(b0, _BTILE)], idx_a, si_a)
        pltpu.async_copy(idx_hbm.at[1, pl.ds(b0, _BTILE)], idx_b, si_b)

        @pl.loop(0, pairs)
        def _(p):
            h_a = 2 * p
            h_b = h_a + 1

            pltpu.make_async_copy(idx_hbm.at[0, pl.ds(b0, _BTILE)],
                                  idx_a, si_a).wait()
            cp_a = pltpu.async_copy(table_hbm.at[idx_a], gath_a, sg_a)

            pltpu.make_async_copy(idx_hbm.at[0, pl.ds(b0, _BTILE)],
                                  idx_b, si_b).wait()
            cp_b = pltpu.async_copy(table_hbm.at[idx_b], gath_b, sg_b)

            cp_a.wait()

            @pl.when(p > 0)
            def _():
                pltpu.make_async_copy(outb_a,
                                      out_hbm.at[0, :, pl.ds(b0, _BTILE)],
                                      so_a).wait()

            transpose_into(gath_a, outb_a)
            pltpu.async_copy(outb_a, out_hbm.at[h_a, :, pl.ds(b0, _BTILE)],
                             so_a)

            @pl.when(p < pairs - 1)
            def _():
                pltpu.async_copy(idx_hbm.at[h_a + 2, pl.ds(b0, _BTILE)],
                                 idx_a, si_a)

            cp_b.wait()

            @pl.when(p > 0)
            def _():
                pltpu.make_async_copy(outb_b,
                                      out_hbm.at[0, :, pl.ds(b0, _BTILE)],
                                      so_b).wait()

            transpose_into(gath_b, outb_b)
            pltpu.async_copy(outb_b, out_hbm.at[h_b, :, pl.ds(b0, _BTILE)],
                             so_b)

            @pl.when(p < pairs - 1)
            def _():
                pltpu.async_copy(idx_hbm.at[h_b + 2, pl.ds(b0, _BTILE)],
                                 idx_b, si_b)

        pltpu.make_async_copy(outb_a, out_hbm.at[0, :, pl.ds(b0, _BTILE)],
                              so_a).wait()
        pltpu.make_async_copy(outb_b, out_hbm.at[0, :, pl.ds(b0, _BTILE)],
                              so_b).wait()

    out = gather_kernel(tab128, idx_t)
    return out.transpose(2, 0, 1)


# speedup vs baseline: 1.3140x; 1.3140x over previous
import dataclasses

import jax
import jax.numpy as jnp
from jax import lax
from jax.experimental import pallas as pl
from jax.experimental.pallas import tpu as pltpu
from jax.experimental.pallas import tpu_sc as plsc

_BTILE = 128
_TILES = 32
_LANE = 128


def kernel(inputs, table):
    batch, hist = inputs.shape
    _, embed_dim = table.shape
    nchunk = embed_dim // 16
    pairs = hist // 2

    tab128 = jnp.pad(table, ((0, 0), (0, _LANE - embed_dim)))
    idx_t = inputs.T

    mesh = plsc.VectorSubcoreMesh(core_axis_name="c", subcore_axis_name="s")

    cp = pltpu.CompilerParams()
    if "needs_layout_passes" in pltpu.CompilerParams.__dataclass_fields__:
        cp = dataclasses.replace(cp, needs_layout_passes=False)

    @pl.kernel(
        out_type=jax.ShapeDtypeStruct((hist, embed_dim, batch), table.dtype),
        mesh=mesh,
        compiler_params=cp,
        scratch_types=[
            pltpu.VMEM((_BTILE,), jnp.int32),
            pltpu.VMEM((_BTILE,), jnp.int32),
            pltpu.VMEM((_BTILE, _LANE), jnp.float32),
            pltpu.VMEM((_BTILE, _LANE), jnp.float32),
            pltpu.VMEM((64, _BTILE), jnp.float32),
            pltpu.VMEM((64, _BTILE), jnp.float32),
            pltpu.SemaphoreType.DMA,
            pltpu.SemaphoreType.DMA,
            pltpu.SemaphoreType.DMA,
            pltpu.SemaphoreType.DMA,
            pltpu.SemaphoreType.DMA,
            pltpu.SemaphoreType.DMA,
        ],
    )
    def gather_kernel(table_hbm, idx_hbm, out_hbm,
                      idx_a, idx_b, gath_a, gath_b, outb_a, outb_b,
                      si_a, si_b, sg_a, sg_b, so_a, so_b):
        wid = lax.axis_index("c") * 16 + lax.axis_index("s")
        b0 = wid * _BTILE

        rows = [lax.iota(jnp.int32, 16) + 16 * j for j in range(_BTILE // 16)]

        def transpose_into(gath, outb):
            @plsc.parallel_loop(0, embed_dim, unroll=4)
            def _(e):
                cols = jnp.full((16,), e, jnp.int32)
                for j in range(_BTILE // 16):
                    x = plsc.load_gather(gath, [rows[j], cols])
                    outb.at[e, pl.ds(16 * j, 16)][...] = x

        pltpu.async_copy(idx_hbm.at[0, pl.ds(b0, _BTILE)], idx_a, si_a)
        pltpu.async_copy(idx_hbm.at[1, pl.ds(b0, _BTILE)], idx_b, si_b)

        @pl.loop(0, pairs)
        def _(p):
            h_a = 2 * p
            h_b = h_a + 1

            pltpu.make_async_copy(idx_hbm.at[0, pl.ds(b0, _BTILE)],
                                  idx_a, si_a).wait()
            cp_a = pltpu.async_copy(table_hbm.at[idx_a], gath_a, sg_a)

            pltpu.make_async_copy(idx_hbm.at[0, pl.ds(b0, _BTILE)],
                                  idx_b, si_b).wait()
            cp_b = pltpu.async_copy(table_hbm.at[idx_b], gath_b, sg_b)

            cp_a.wait()

            @pl.when(p > 0)
            def _():
                pltpu.make_async_copy(outb_a,
                                      out_hbm.at[0, :, pl.ds(b0, _BTILE)],
                                      so_a).wait()

            transpose_into(gath_a, outb_a)
            pltpu.async_copy(outb_a, out_hbm.at[h_a, :, pl.ds(b0, _BTILE)],
                             so_a)

            @pl.when(p < pairs - 1)
            def _():
                pltpu.async_copy(idx_hbm.at[h_a + 2, pl.ds(b0, _BTILE)],
                                 idx_a, si_a)

            cp_b.wait()

            @pl.when(p > 0)
            def _():
                pltpu.make_async_copy(outb_b,
                                      out_hbm.at[0, :, pl.ds(b0, _BTILE)],
                                      so_b).wait()

            transpose_into(gath_b, outb_b)
            pltpu.async_copy(outb_b, out_hbm.at[h_b, :, pl.ds(b0, _BTILE)],
                             so_b)

            @pl.when(p < pairs - 1)
            def _():
                pltpu.async_copy(idx_hbm.at[h_b + 2, pl.ds(b0, _BTILE)],
                                 idx_b, si_b)

        pltpu.make_async_copy(outb_a, out_hbm.at[0, :, pl.ds(b0, _BTILE)],
                              so_a).wait()
        pltpu.make_async_copy(outb_b, out_hbm.at[0, :, pl.ds(b0, _BTILE)],
                              so_b).wait()

    out = gather_kernel(tab128, idx_t)
    return out.transpose(2, 0, 1)


# speedup vs baseline: 2.0665x; 1.5726x over previous
import dataclasses

import jax
import jax.numpy as jnp
from jax import lax
from jax.experimental import pallas as pl
from jax.experimental.pallas import tpu as pltpu
from jax.experimental.pallas import tpu_sc as plsc

_BTILE = 128
_TILES = 32
_LANE = 128


def kernel(inputs, table):
    batch, hist = inputs.shape
    _, embed_dim = table.shape
    nchunk = embed_dim // 16
    pairs = hist // 2

    tab128 = jnp.pad(table, ((0, 0), (0, _LANE - embed_dim)))
    idx_t = inputs.T

    mesh = plsc.VectorSubcoreMesh(core_axis_name="c", subcore_axis_name="s")

    cp = pltpu.CompilerParams()
    if "needs_layout_passes" in pltpu.CompilerParams.__dataclass_fields__:
        cp = dataclasses.replace(cp, needs_layout_passes=False)

    @pl.kernel(
        out_type=jax.ShapeDtypeStruct((hist, embed_dim, batch), table.dtype),
        mesh=mesh,
        compiler_params=cp,
        scratch_types=[
            pltpu.VMEM((_BTILE,), jnp.int32),
            pltpu.VMEM((_BTILE,), jnp.int32),
            pltpu.VMEM((_BTILE, _LANE), jnp.float32),
            pltpu.VMEM((_BTILE, _LANE), jnp.float32),
            pltpu.VMEM((64, _BTILE), jnp.float32),
            pltpu.VMEM((64, _BTILE), jnp.float32),
            pltpu.SemaphoreType.DMA,
            pltpu.SemaphoreType.DMA,
            pltpu.SemaphoreType.DMA,
            pltpu.SemaphoreType.DMA,
            pltpu.SemaphoreType.DMA,
            pltpu.SemaphoreType.DMA,
        ],
    )
    def gather_kernel(table_hbm, idx_hbm, out_hbm,
                      idx_a, idx_b, gath_a, gath_b, outb_a, outb_b,
                      si_a, si_b, sg_a, sg_b, so_a, so_b):
        wid = lax.axis_index("c") * 16 + lax.axis_index("s")
        b0 = wid * _BTILE

        iota = lax.iota(jnp.int32, 16)
        diag = [(iota + k) & 15 for k in range(16)]
        rows = [iota + 16 * j for j in range(_BTILE // 16)]

        def transpose_into(gath, outb):
            @plsc.parallel_loop(0, nchunk, unroll=2)
            def _(c):
                e0 = 16 * c
                for k in range(16):
                    cols = e0 + diag[k]
                    for j in range(_BTILE // 16):
                        x = plsc.load_gather(gath, [rows[j], cols])
                        plsc.store_scatter(outb, [cols, rows[j]], x)

        pltpu.async_copy(idx_hbm.at[0, pl.ds(b0, _BTILE)], idx_a, si_a)
        pltpu.async_copy(idx_hbm.at[1, pl.ds(b0, _BTILE)], idx_b, si_b)

        @pl.loop(0, pairs)
        def _(p):
            h_a = 2 * p
            h_b = h_a + 1

            pltpu.make_async_copy(idx_hbm.at[0, pl.ds(b0, _BTILE)],
                                  idx_a, si_a).wait()
            cp_a = pltpu.async_copy(table_hbm.at[idx_a], gath_a, sg_a)

            pltpu.make_async_copy(idx_hbm.at[0, pl.ds(b0, _BTILE)],
                                  idx_b, si_b).wait()
            cp_b = pltpu.async_copy(table_hbm.at[idx_b], gath_b, sg_b)

            cp_a.wait()

            @pl.when(p > 0)
            def _():
                pltpu.make_async_copy(outb_a,
                                      out_hbm.at[0, :, pl.ds(b0, _BTILE)],
                                      so_a).wait()

            transpose_into(gath_a, outb_a)
            pltpu.async_copy(outb_a, out_hbm.at[h_a, :, pl.ds(b0, _BTILE)],
                             so_a)

            @pl.when(p < pairs - 1)
            def _():
                pltpu.async_copy(idx_hbm.at[h_a + 2, pl.ds(b0, _BTILE)],
                                 idx_a, si_a)

            cp_b.wait()

            @pl.when(p > 0)
            def _():
                pltpu.make_async_copy(outb_b,
                                      out_hbm.at[0, :, pl.ds(b0, _BTILE)],
                                      so_b).wait()

            transpose_into(gath_b, outb_b)
            pltpu.async_copy(outb_b, out_hbm.at[h_b, :, pl.ds(b0, _BTILE)],
                             so_b)

            @pl.when(p < pairs - 1)
            def _():
                pltpu.async_copy(idx_hbm.at[h_b + 2, pl.ds(b0, _BTILE)],
                                 idx_b, si_b)

        pltpu.make_async_copy(outb_a, out_hbm.at[0, :, pl.ds(b0, _BTILE)],
                              so_a).wait()
        pltpu.make_async_copy(outb_b, out_hbm.at[0, :, pl.ds(b0, _BTILE)],
                              so_b).wait()

    out = gather_kernel(tab128, idx_t)
    return out.transpose(2, 0, 1)
